# baseline (speedup 1.0000x reference)
import functools

import jax
import jax.numpy as jnp
from jax import lax
from jax.experimental import pallas as pl
from jax.experimental.pallas import tpu as pltpu
from jax.experimental.pallas import tpu_sc as plsc

B = 4096
NC = 2
NS = 16
NW = NC * NS
BPW = B // NW
L = 16
DB = 10
DG = 3
DP = 69
DT = 3
N = 100000


def _worker_base():
    wid = lax.axis_index("s") * NC + lax.axis_index("c")
    return wid * BPW


def _body_narrow(idx_hbm, betas_hbm, go_hbm, tr_hbm,
                 out_b, out_go, out_tr,
                 idx_v, ego, bet_v, b_rows, go_rows, tr_rows,
                 sem, osem):
    base = _worker_base()
    pltpu.sync_copy(idx_hbm.at[pl.ds(base, BPW)], idx_v)
    pltpu.sync_copy(betas_hbm, bet_v)

    iota = lax.iota(jnp.int32, L)
    zeros = jnp.zeros((L,), jnp.int32)
    jconst0 = iota * N

    def row(b, _):
        ivec = plsc.load_gather(idx_v, [jnp.full((L,), b, jnp.int32)])
        plsc.store_scatter(ego, [DG * b + iota], jconst0 + ivec)
        return _

    lax.fori_loop(0, BPW, row, None)

    cps = [
        pltpu.async_copy(go_hbm.at[ego.at[pl.ds(0, BPW * DG)]], go_rows, sem),
        pltpu.async_copy(tr_hbm.at[ego.at[pl.ds(0, BPW * DG)]], tr_rows, sem),
    ]

    for m in range(5):
        lane = iota + 16 * m
        col = lane
        rsub = zeros
        for t in (10, 20, 30, 40, 50, 60, 70):
            col = jnp.where(lane >= t, lane - t, col)
            rsub = rsub + jnp.where(lane >= t, 1, 0)
        vm = plsc.load_gather(bet_v, [zeros, col])
        for r in range(BPW * DB // 80):
            plsc.store_scatter(b_rows, [8 * r + rsub, col], vm)

    for cp in cps:
        cp.wait()

    ocps = [
        pltpu.async_copy(go_rows, out_go.at[pl.ds(base * DG, BPW * DG)], osem),
        pltpu.async_copy(tr_rows, out_tr.at[pl.ds(base * DT, BPW * DT)], osem),
        pltpu.async_copy(b_rows, out_b.at[pl.ds(base, BPW)], osem),
    ]
    for cp in ocps:
        cp.wait()


def _make_body_wide(d):
    nk = (d + L - 1) // L

    def _body_wide(idx_hbm, bp_hbm, out_bp, idx_v, ebp, bp_rows, sem, osem):
        base = _worker_base()
        pltpu.sync_copy(idx_hbm.at[pl.ds(base, BPW)], idx_v)

        iota = lax.iota(jnp.int32, L)
        jconst = [(16 * k + iota) * N for k in range(nk)]

        def row(b, _):
            ivec = plsc.load_gather(idx_v, [jnp.full((L,), b, jnp.int32)])
            for k in range(nk):
                plsc.store_scatter(ebp, [d * b + 16 * k + iota],
                                   jconst[k] + ivec)
            return _

        NGRP = 4
        RG = BPW // NGRP
        EG = RG * d
        cps = []
        for g in range(NGRP):
            lax.fori_loop(g * RG, (g + 1) * RG, row, None)
            cps.append(pltpu.async_copy(
                bp_hbm.at[ebp.at[pl.ds(g * EG, EG)]],
                bp_rows.at[pl.ds(g * EG, EG)], sem))
        for cp in cps:
            cp.wait()
        pltpu.async_copy(bp_rows, out_bp.at[pl.ds(base * d, BPW * d)],
                         osem).wait()

    return _body_wide


def kernel(idx, betas, global_orient, body_pose, transl):
    idx = idx.astype(jnp.int32)
    go_f = global_orient.T.reshape(-1)
    tr_f = transl.T.reshape(-1)
    mesh = plsc.VectorSubcoreMesh(core_axis_name="c", subcore_axis_name="s")
    cp = pltpu.CompilerParams(needs_layout_passes=False)

    run_narrow = functools.partial(
        pl.kernel,
        mesh=mesh,
        compiler_params=cp,
        out_type=[
            jax.ShapeDtypeStruct((B, DB), jnp.float32),
            jax.ShapeDtypeStruct((B * DG,), jnp.float32),
            jax.ShapeDtypeStruct((B * DT,), jnp.float32),
        ],
        scratch_types=[
            pltpu.VMEM((BPW,), jnp.int32),
            pltpu.VMEM((BPW * DG + 16,), jnp.int32),
            pltpu.VMEM((1, DB), jnp.float32),
            pltpu.VMEM((BPW, DB), jnp.float32),
            pltpu.VMEM((BPW * DG,), jnp.float32),
            pltpu.VMEM((BPW * DT,), jnp.float32),
            pltpu.SemaphoreType.DMA,
            pltpu.SemaphoreType.DMA,
        ],
    )(_body_narrow)

    def make_wide(d):
        return functools.partial(
            pl.kernel,
            mesh=mesh,
            compiler_params=cp,
            out_type=[jax.ShapeDtypeStruct((B * d,), jnp.float32)],
            scratch_types=[
                pltpu.VMEM((BPW,), jnp.int32),
                pltpu.VMEM((BPW * d + 16,), jnp.int32),
                pltpu.VMEM((BPW * d,), jnp.float32),
                pltpu.SemaphoreType.DMA,
                pltpu.SemaphoreType.DMA,
            ],
        )(_make_body_wide(d))

    DL = 53
    DR = DP - DL
    bp_l = body_pose[:, :DL].T.reshape(-1)
    bp_r = body_pose[:, DL:].T.reshape(-1)

    ob, ogo, otr = run_narrow(idx, betas, go_f, tr_f)
    (obp_l,) = make_wide(DL)(idx, bp_l)
    (obp_r,) = make_wide(DR)(idx, bp_r)
    obp = jnp.concatenate(
        [obp_l.reshape(B, DL), obp_r.reshape(B, DR)], axis=1)
    return (ob, ogo.reshape(B, DG), obp, otr.reshape(B, DT))
